# baseline (speedup 1.0000x reference)
import jax
import jax.numpy as jnp
from jax.experimental import pallas as pl
from jax.experimental.pallas import tpu as pltpu

_NUM_CLASSES = 8192
_Z_DIM = 256
_BLOCK_ROWS = 8192


def _copy_body(a_ref, o_ref):
    o_ref[...] = a_ref[...]


def kernel(_, anchor):
    grid = (_NUM_CLASSES // _BLOCK_ROWS,)
    return pl.pallas_call(
        _copy_body,
        grid=grid,
        in_specs=[pl.BlockSpec((_BLOCK_ROWS, _Z_DIM), lambda i: (i, 0))],
        out_specs=pl.BlockSpec((_BLOCK_ROWS, _Z_DIM), lambda i: (i, 0)),
        out_shape=jax.ShapeDtypeStruct((_NUM_CLASSES, _Z_DIM), jnp.float32),
        compiler_params=pltpu.CompilerParams(dimension_semantics=("parallel",)),
    )(anchor)


# speedup vs baseline: 1.2089x; 1.2089x over previous
import jax
import jax.numpy as jnp
from jax.experimental import pallas as pl
from jax.experimental.pallas import tpu as pltpu

_NUM_CLASSES = 8192
_Z_DIM = 256
_BLOCK_ROWS = 4096


def _copy_body(a_ref, o_ref):
    o_ref[...] = a_ref[...]


def kernel(_, anchor):
    grid = (_NUM_CLASSES // _BLOCK_ROWS,)
    return pl.pallas_call(
        _copy_body,
        grid=grid,
        in_specs=[pl.BlockSpec((_BLOCK_ROWS, _Z_DIM), lambda i: (i, 0))],
        out_specs=pl.BlockSpec((_BLOCK_ROWS, _Z_DIM), lambda i: (i, 0)),
        out_shape=jax.ShapeDtypeStruct((_NUM_CLASSES, _Z_DIM), jnp.float32),
        compiler_params=pltpu.CompilerParams(dimension_semantics=("arbitrary",)),
    )(anchor)
